# Initial kernel scaffold; baseline (speedup 1.0000x reference)
import jax
import jax.numpy as jnp
from jax.experimental import pallas as pl


def kernel(x, W_enc, W_dec):
    raise NotImplementedError("write your pallas kernel here")


# speedup vs baseline: 5.8960x; 5.8960x over previous
import jax
import jax.numpy as jnp
from jax.experimental import pallas as pl
from jax.experimental.pallas import tpu as pltpu

_N, _DI, _DL, _K = 8192, 2048, 16384, 32

_BT_E, _BL_E = 512, 1024


def _enc_body(x_ref, w_ref, o_ref):
    acc = jax.lax.dot_general(
        x_ref[...], w_ref[...], (((1,), (1,)), ((), ())),
        preferred_element_type=jnp.float32,
        precision=jax.lax.Precision.DEFAULT)
    o_ref[...] = jnp.maximum(acc, 0.0)


def _encode(x, W_enc):
    nj, ni = _DL // _BL_E, _N // _BT_E
    return pl.pallas_call(
        _enc_body,
        grid=(nj, ni),
        in_specs=[
            pl.BlockSpec((_BT_E, _DI), lambda j, i: (i, 0)),
            pl.BlockSpec((_BL_E, _DI), lambda j, i: (j, 0)),
        ],
        out_specs=pl.BlockSpec((_BT_E, _BL_E), lambda j, i: (i, j)),
        out_shape=jax.ShapeDtypeStruct((_N, _DL), jnp.float32),
        compiler_params=pltpu.CompilerParams(
            dimension_semantics=("arbitrary", "arbitrary")),
    )(x, W_enc)


_BT_T = 128


def _topk_body(z_ref, o_ref, scratch_ref):
    scratch_ref[...] = z_ref[...]

    def step(_, t):
        m = jnp.max(scratch_ref[...], axis=1, keepdims=True)
        scratch_ref[...] = jnp.where(scratch_ref[...] == m, -jnp.inf,
                                     scratch_ref[...])
        return m

    t = jax.lax.fori_loop(0, _K, step,
                          jnp.zeros((_BT_T, 1), jnp.float32))
    z = z_ref[...]
    o_ref[...] = jnp.where(z >= t, z, 0.0)


def _topk_mask(z):
    return pl.pallas_call(
        _topk_body,
        grid=(_N // _BT_T,),
        in_specs=[pl.BlockSpec((_BT_T, _DL), lambda i: (i, 0))],
        out_specs=pl.BlockSpec((_BT_T, _DL), lambda i: (i, 0)),
        out_shape=jax.ShapeDtypeStruct((_N, _DL), jnp.float32),
        scratch_shapes=[pltpu.VMEM((_BT_T, _DL), jnp.float32)],
        compiler_params=pltpu.CompilerParams(
            dimension_semantics=("arbitrary",)),
    )(z)


_BT_D, _BK_D = 512, 1024


def _dec_body(z_ref, w_ref, o_ref):
    zb = z_ref[...].astype(jnp.bfloat16)
    wb = w_ref[...].astype(jnp.bfloat16)
    part = jax.lax.dot_general(
        zb, wb, (((1,), (1,)), ((), ())),
        preferred_element_type=jnp.float32)

    @pl.when(pl.program_id(1) == 0)
    def _init():
        o_ref[...] = part

    @pl.when(pl.program_id(1) != 0)
    def _acc():
        o_ref[...] += part


def _decode(z_masked, W_dec):
    ni, nk = _N // _BT_D, _DL // _BK_D
    return pl.pallas_call(
        _dec_body,
        grid=(ni, nk),
        in_specs=[
            pl.BlockSpec((_BT_D, _BK_D), lambda i, k: (i, k)),
            pl.BlockSpec((_DI, _BK_D), lambda i, k: (0, k)),
        ],
        out_specs=pl.BlockSpec((_BT_D, _DI), lambda i, k: (i, 0)),
        out_shape=jax.ShapeDtypeStruct((_N, _DI), jnp.float32),
        compiler_params=pltpu.CompilerParams(
            dimension_semantics=("arbitrary", "arbitrary")),
    )(z_masked, W_dec)


def kernel(x, W_enc, W_dec):
    z = _encode(x, W_enc)
    z_masked = _topk_mask(z)
    x_hat = _decode(z_masked, W_dec)
    return (x_hat, z_masked)
